# baseline (speedup 1.0000x reference)
import functools

import jax
import jax.numpy as jnp
from jax import lax
from jax.experimental import pallas as pl
from jax.experimental.pallas import tpu as pltpu
from jax.experimental.pallas import tpu_sc as plsc

_X = 1000000
_YC = 32
_YCP = 40
_N = 819200
_LO = -1.0

_NC, _NS, _L = 2, 16, 16
_NW = _NC * _NS
_QW = _N // _NW
_CHUNK = 1024
_NCHUNK = _QW // _CHUNK
_BQ = 128
_NB = _CHUNK // _BQ
_NG = _BQ // _L

_AW = 256
_AFULL = 3904
_APW = _AFULL // _NW
_ATAIL0 = _AFULL * _AW
_AT1 = 256
_AT2 = _X - _ATAIL0 - 2 * _AT1


def _transpose_pad(chebT, cheb_tail):
    mesh = plsc.VectorSubcoreMesh(core_axis_name="c", subcore_axis_name="s")

    @functools.partial(
        pl.kernel,
        out_type=jax.ShapeDtypeStruct((_X * _YCP,), jnp.float32),
        mesh=mesh,
        compiler_params=pltpu.CompilerParams(
            needs_layout_passes=False, use_tc_tiling_on_sc=True),
        scratch_types=[
            pltpu.VMEM((_YC, _AW), jnp.float32),
            pltpu.VMEM((_YC, _AW), jnp.float32),
            pltpu.VMEM((_AW * _YCP,), jnp.float32),
            pltpu.VMEM((_AW * _YCP,), jnp.float32),
            pltpu.VMEM((_AT2, _YC), jnp.float32),
            pltpu.SemaphoreType.DMA,
            pltpu.SemaphoreType.DMA,
            pltpu.SemaphoreType.DMA,
            pltpu.SemaphoreType.DMA,
        ],
    )
    def transpose_pad(t_hbm, tail_hbm, out_hbm, cin0, cin1, cout0, cout1,
                      tin, semr0, semr1, semw0, semw1):
        wid = lax.axis_index("s") * _NC + lax.axis_index("c")
        cins, couts = (cin0, cin1), (cout0, cout1)
        semrs, semws = (semr0, semr1), (semw0, semw1)
        lane40 = lax.iota(jnp.int32, _L) * _YCP

        def col0(i):
            return (i * _NW + wid) * _AW

        def read(i, u, start):
            mk = pltpu.async_copy if start else pltpu.make_async_copy
            return mk(t_hbm.at[:, pl.ds(col0(i), _AW)], cins[u], semrs[u])

        def write(i, u, start):
            mk = pltpu.async_copy if start else pltpu.make_async_copy
            return mk(
                couts[u],
                out_hbm.at[pl.ds(col0(i) * _YCP, _AW * _YCP)], semws[u])

        def scatter(cin, cout, width):
            def g_body(g, carry):
                for n in range(_YC):
                    v = cin[n, pl.ds(g * _L, _L)]
                    plsc.store_scatter(
                        cout, [lane40 + (g * (_L * _YCP) + n)], v)
                return carry
            lax.fori_loop(0, width // _L, g_body, 0)

        def step(i, u):
            read(i, u, start=False).wait()

            @pl.when(i >= 2)
            def _():
                write(i - 2, u, start=False).wait()

            scatter(cins[u], couts[u], _AW)
            write(i, u, start=True)

            @pl.when(i + 2 < _APW)
            def _():
                read(i + 2, u, start=True)

        read(0, 0, start=True)
        read(1, 1, start=True)

        def pair(j, carry):
            step(2 * j, 0)
            step(2 * j + 1, 1)
            return carry

        lax.fori_loop(0, _APW // 2, pair, 0)
        write(_APW - 2, 0, start=False).wait()
        write(_APW - 1, 1, start=False).wait()

        for w in (0, 1):
            @pl.when(wid == w)
            def _(w=w):
                c0 = _ATAIL0 + w * _AT1
                pltpu.sync_copy(t_hbm.at[:, pl.ds(c0, _AT1)], cin0)
                scatter(cin0, cout0, _AT1)
                pltpu.sync_copy(
                    cout0, out_hbm.at[pl.ds(c0 * _YCP, _AT1 * _YCP)])

        @pl.when(wid == 2)
        def _():
            pltpu.sync_copy(tail_hbm, tin)

            def r_body(r, carry):
                cout1[pl.ds(r * _YCP, _L)] = tin[r, pl.ds(0, _L)]
                cout1[pl.ds(r * _YCP + _L, _L)] = tin[r, pl.ds(_L, _L)]
                return carry
            lax.fori_loop(0, _AT2, r_body, 0)
            c0 = _ATAIL0 + 2 * _AT1
            pltpu.sync_copy(
                cout1.at[pl.ds(0, _AT2 * _YCP)],
                out_hbm.at[pl.ds(c0 * _YCP, _AT2 * _YCP)])

    return transpose_pad(chebT, cheb_tail)


def _series_eval(z, table):
    mesh = plsc.VectorSubcoreMesh(core_axis_name="c", subcore_axis_name="s")

    @functools.partial(
        pl.kernel,
        out_type=jax.ShapeDtypeStruct((_N,), jnp.float32),
        mesh=mesh,
        compiler_params=pltpu.CompilerParams(
            needs_layout_passes=False, use_tc_tiling_on_sc=False),
        scratch_types=[
            [pltpu.VMEM((_CHUNK,), jnp.float32)] * 2,
            [pltpu.VMEM((_NB, _BQ), jnp.int32)] * 2,
            [pltpu.VMEM((_CHUNK,), jnp.float32)] * 2,
            [pltpu.VMEM((_NB, _BQ, _YCP), jnp.float32)] * 2,
            [pltpu.VMEM((_CHUNK,), jnp.float32)] * 2,
            [pltpu.SemaphoreType.DMA] * 2,
            [pltpu.SemaphoreType.DMA] * 2,
            [pltpu.SemaphoreType.DMA] * 2,
        ],
    )
    def run(z_hbm, cheb_hbm, out_hbm, z_v, idx_v, y_v, rows_v, out_v,
            semz, semg, semw):
        wid = lax.axis_index("s") * _NC + lax.axis_index("c")
        base = wid * _QW

        def zread(c, u, start):
            mk = pltpu.async_copy if start else pltpu.make_async_copy
            return mk(
                z_hbm.at[pl.ds(base + c * _CHUNK, _CHUNK)], z_v[u], semz[u])

        def gather(c, b, u, start):
            mk = pltpu.async_copy if start else pltpu.make_async_copy
            return mk(
                cheb_hbm.at[idx_v[u].at[b]], rows_v[u].at[b], semg[u])

        def outwrite(c, u, start):
            mk = pltpu.async_copy if start else pltpu.make_async_copy
            return mk(
                out_v[u], out_hbm.at[pl.ds(base + c * _CHUNK, _CHUNK)],
                semw[u])

        def stage_a(c, u):
            zread(c, u, start=False).wait()
            for i in range(_CHUNK // _L):
                t = z_v[u][pl.ds(i * _L, _L)] - _LO
                xi = (t * 0.5).astype(jnp.int32)
                xi = jnp.minimum(xi, _X - 1)
                y = t - 2.0 * xi.astype(jnp.float32) + _LO
                y = jnp.minimum(jnp.maximum(y, -1.0 + 1e-6), 1.0 - 1e-6)
                idx_v[u][i // _NG, pl.ds((i % _NG) * _L, _L)] = xi
                y_v[u][pl.ds(i * _L, _L)] = y
            for b in range(_NB):
                gather(c, b, u, start=True)

            @pl.when(c + 2 < _NCHUNK)
            def _():
                zread(c + 2, u, start=True)

        def stage_b(c, u):
            @pl.when(c >= 2)
            def _():
                outwrite(c - 2, u, start=False).wait()

            _NI = 4
            for b in range(_NB):
                gather(c, b, u, start=False).wait()
                rows_b = rows_v[u].at[b]

                def group_body(gg, _, b=b, rows_b=rows_b):
                    gs = [gg * _NI + k for k in range(_NI)]
                    qidx = [lax.iota(jnp.int32, _L) + g * _L for g in gs]
                    y = [y_v[u][pl.ds(b * _BQ + g * _L, _L)] for g in gs]
                    y2 = [yk + yk for yk in y]

                    def ld(k, n):
                        return plsc.load_gather(
                            rows_b, [qidx[k], jnp.full((_L,), n, jnp.int32)])

                    bk1 = [ld(k, _YC - 1) for k in range(_NI)]
                    bk2 = [jnp.zeros((_L,), jnp.float32)] * _NI
                    for n in range(_YC - 2, 0, -1):
                        a = [ld(k, n) for k in range(_NI)]
                        for k in range(_NI):
                            bk1[k], bk2[k] = (
                                a[k] + y2[k] * bk1[k] - bk2[k], bk1[k])
                    for k in range(_NI):
                        a0 = ld(k, 0)
                        out_v[u][pl.ds(b * _BQ + gs[k] * _L, _L)] = (
                            a0 + y[k] * bk1[k] - bk2[k])
                    return _

                lax.fori_loop(0, _NG // _NI, group_body, 0)

            outwrite(c, u, start=True)

        zread(0, 0, start=True)
        zread(1, 1, start=True)
        stage_a(0, 0)

        def pair(j, carry):
            c = 2 * j
            stage_a(c + 1, 1)
            stage_b(c, 0)

            @pl.when(c + 2 < _NCHUNK)
            def _():
                stage_a(c + 2, 0)

            stage_b(c + 1, 1)
            return carry

        lax.fori_loop(0, _NCHUNK // 2, pair, 0)
        if _NCHUNK % 2:
            stage_b(_NCHUNK - 1, (_NCHUNK - 1) % 2)
        outwrite(_NCHUNK - 2, (_NCHUNK - 2) % 2, start=False).wait()
        outwrite(_NCHUNK - 1, (_NCHUNK - 1) % 2, start=False).wait()

    return run(z, table)


def kernel(z, cheb):
    chebT = lax.optimization_barrier(cheb.T)
    flat = _transpose_pad(chebT, cheb[_X - _AT2:])
    return _series_eval(z, flat.reshape(_X, _YCP))


# speedup vs baseline: 1.0449x; 1.0449x over previous
import functools

import jax
import jax.numpy as jnp
from jax import lax
from jax.experimental import pallas as pl
from jax.experimental.pallas import tpu as pltpu
from jax.experimental.pallas import tpu_sc as plsc

_X = 1000000
_YC = 32
_YCP = 40
_N = 819200
_LO = -1.0

_NC, _NS, _L = 2, 16, 16
_NW = _NC * _NS
_QW = _N // _NW
_CHUNK = 512
_NCHUNK = _QW // _CHUNK
_BQ = 128
_NB = _CHUNK // _BQ
_NG = _BQ // _L

_AW = 256
_AFULL = 3904
_APW = _AFULL // _NW
_ATAIL0 = _AFULL * _AW
_AT1 = 256
_AT2 = _X - _ATAIL0 - 2 * _AT1


def _transpose_pad(chebT, cheb_tail):
    mesh = plsc.VectorSubcoreMesh(core_axis_name="c", subcore_axis_name="s")

    @functools.partial(
        pl.kernel,
        out_type=jax.ShapeDtypeStruct((_X * _YCP,), jnp.float32),
        mesh=mesh,
        compiler_params=pltpu.CompilerParams(
            needs_layout_passes=False, use_tc_tiling_on_sc=True),
        scratch_types=[
            pltpu.VMEM((_YC, _AW), jnp.float32),
            pltpu.VMEM((_YC, _AW), jnp.float32),
            pltpu.VMEM((_AW * _YCP,), jnp.float32),
            pltpu.VMEM((_AW * _YCP,), jnp.float32),
            pltpu.VMEM((_AT2, _YC), jnp.float32),
            pltpu.SemaphoreType.DMA,
            pltpu.SemaphoreType.DMA,
            pltpu.SemaphoreType.DMA,
            pltpu.SemaphoreType.DMA,
        ],
    )
    def transpose_pad(t_hbm, tail_hbm, out_hbm, cin0, cin1, cout0, cout1,
                      tin, semr0, semr1, semw0, semw1):
        wid = lax.axis_index("s") * _NC + lax.axis_index("c")
        cins, couts = (cin0, cin1), (cout0, cout1)
        semrs, semws = (semr0, semr1), (semw0, semw1)
        lane40 = lax.iota(jnp.int32, _L) * _YCP

        def col0(i):
            return (i * _NW + wid) * _AW

        def read(i, u, start):
            mk = pltpu.async_copy if start else pltpu.make_async_copy
            return mk(t_hbm.at[:, pl.ds(col0(i), _AW)], cins[u], semrs[u])

        def write(i, u, start):
            mk = pltpu.async_copy if start else pltpu.make_async_copy
            return mk(
                couts[u],
                out_hbm.at[pl.ds(col0(i) * _YCP, _AW * _YCP)], semws[u])

        def scatter(cin, cout, width):
            def g_body(g, carry):
                for n in range(_YC):
                    v = cin[n, pl.ds(g * _L, _L)]
                    plsc.store_scatter(
                        cout, [lane40 + (g * (_L * _YCP) + n)], v)
                return carry
            lax.fori_loop(0, width // _L, g_body, 0)

        def step(i, u):
            read(i, u, start=False).wait()

            @pl.when(i >= 2)
            def _():
                write(i - 2, u, start=False).wait()

            scatter(cins[u], couts[u], _AW)
            write(i, u, start=True)

            @pl.when(i + 2 < _APW)
            def _():
                read(i + 2, u, start=True)

        read(0, 0, start=True)
        read(1, 1, start=True)

        def pair(j, carry):
            step(2 * j, 0)
            step(2 * j + 1, 1)
            return carry

        lax.fori_loop(0, _APW // 2, pair, 0)
        write(_APW - 2, 0, start=False).wait()
        write(_APW - 1, 1, start=False).wait()

        for w in (0, 1):
            @pl.when(wid == w)
            def _(w=w):
                c0 = _ATAIL0 + w * _AT1
                pltpu.sync_copy(t_hbm.at[:, pl.ds(c0, _AT1)], cin0)
                scatter(cin0, cout0, _AT1)
                pltpu.sync_copy(
                    cout0, out_hbm.at[pl.ds(c0 * _YCP, _AT1 * _YCP)])

        @pl.when(wid == 2)
        def _():
            pltpu.sync_copy(tail_hbm, tin)

            def r_body(r, carry):
                cout1[pl.ds(r * _YCP, _L)] = tin[r, pl.ds(0, _L)]
                cout1[pl.ds(r * _YCP + _L, _L)] = tin[r, pl.ds(_L, _L)]
                return carry
            lax.fori_loop(0, _AT2, r_body, 0)
            c0 = _ATAIL0 + 2 * _AT1
            pltpu.sync_copy(
                cout1.at[pl.ds(0, _AT2 * _YCP)],
                out_hbm.at[pl.ds(c0 * _YCP, _AT2 * _YCP)])

    return transpose_pad(chebT, cheb_tail)


def _series_eval(z, table):
    mesh = plsc.VectorSubcoreMesh(core_axis_name="c", subcore_axis_name="s")

    @functools.partial(
        pl.kernel,
        out_type=jax.ShapeDtypeStruct((_N,), jnp.float32),
        mesh=mesh,
        compiler_params=pltpu.CompilerParams(
            needs_layout_passes=False, use_tc_tiling_on_sc=False),
        scratch_types=[
            [pltpu.VMEM((_CHUNK,), jnp.float32)] * 2,
            [pltpu.VMEM((_NB, _BQ), jnp.int32)] * 2,
            [pltpu.VMEM((_CHUNK,), jnp.float32)] * 2,
            [pltpu.VMEM((_NB, _BQ, _YCP), jnp.float32)] * 2,
            [pltpu.VMEM((_CHUNK,), jnp.float32)] * 2,
            [pltpu.SemaphoreType.DMA] * 2,
            [pltpu.SemaphoreType.DMA] * 2,
            [pltpu.SemaphoreType.DMA] * 2,
        ],
    )
    def run(z_hbm, cheb_hbm, out_hbm, z_v, idx_v, y_v, rows_v, out_v,
            semz, semg, semw):
        wid = lax.axis_index("s") * _NC + lax.axis_index("c")
        base = wid * _QW

        def zread(c, u, start):
            mk = pltpu.async_copy if start else pltpu.make_async_copy
            return mk(
                z_hbm.at[pl.ds(base + c * _CHUNK, _CHUNK)], z_v[u], semz[u])

        def gather(c, b, u, start):
            mk = pltpu.async_copy if start else pltpu.make_async_copy
            return mk(
                cheb_hbm.at[idx_v[u].at[b]], rows_v[u].at[b], semg[u])

        def outwrite(c, u, start):
            mk = pltpu.async_copy if start else pltpu.make_async_copy
            return mk(
                out_v[u], out_hbm.at[pl.ds(base + c * _CHUNK, _CHUNK)],
                semw[u])

        def stage_a(c, u):
            zread(c, u, start=False).wait()
            for i in range(_CHUNK // _L):
                t = z_v[u][pl.ds(i * _L, _L)] - _LO
                xi = (t * 0.5).astype(jnp.int32)
                xi = jnp.minimum(xi, _X - 1)
                y = t - 2.0 * xi.astype(jnp.float32) + _LO
                y = jnp.minimum(jnp.maximum(y, -1.0 + 1e-6), 1.0 - 1e-6)
                idx_v[u][i // _NG, pl.ds((i % _NG) * _L, _L)] = xi
                y_v[u][pl.ds(i * _L, _L)] = y
            for b in range(_NB):
                gather(c, b, u, start=True)

            @pl.when(c + 2 < _NCHUNK)
            def _():
                zread(c + 2, u, start=True)

        def stage_b(c, u):
            @pl.when(c >= 2)
            def _():
                outwrite(c - 2, u, start=False).wait()

            _NI = 4
            for b in range(_NB):
                gather(c, b, u, start=False).wait()
                rows_b = rows_v[u].at[b]

                def group_body(gg, _, b=b, rows_b=rows_b):
                    gs = [gg * _NI + k for k in range(_NI)]
                    qidx = [lax.iota(jnp.int32, _L) + g * _L for g in gs]
                    y = [y_v[u][pl.ds(b * _BQ + g * _L, _L)] for g in gs]
                    y2 = [yk + yk for yk in y]

                    def ld(k, n):
                        return plsc.load_gather(
                            rows_b, [qidx[k], jnp.full((_L,), n, jnp.int32)])

                    bk1 = [ld(k, _YC - 1) for k in range(_NI)]
                    bk2 = [jnp.zeros((_L,), jnp.float32)] * _NI
                    for n in range(_YC - 2, 0, -1):
                        a = [ld(k, n) for k in range(_NI)]
                        for k in range(_NI):
                            bk1[k], bk2[k] = (
                                a[k] + y2[k] * bk1[k] - bk2[k], bk1[k])
                    for k in range(_NI):
                        a0 = ld(k, 0)
                        out_v[u][pl.ds(b * _BQ + gs[k] * _L, _L)] = (
                            a0 + y[k] * bk1[k] - bk2[k])
                    return _

                lax.fori_loop(0, _NG // _NI, group_body, 0)

            outwrite(c, u, start=True)

        zread(0, 0, start=True)
        zread(1, 1, start=True)
        stage_a(0, 0)

        def pair(j, carry):
            c = 2 * j
            stage_a(c + 1, 1)
            stage_b(c, 0)

            @pl.when(c + 2 < _NCHUNK)
            def _():
                stage_a(c + 2, 0)

            stage_b(c + 1, 1)
            return carry

        lax.fori_loop(0, _NCHUNK // 2, pair, 0)
        if _NCHUNK % 2:
            stage_b(_NCHUNK - 1, (_NCHUNK - 1) % 2)
        outwrite(_NCHUNK - 2, (_NCHUNK - 2) % 2, start=False).wait()
        outwrite(_NCHUNK - 1, (_NCHUNK - 1) % 2, start=False).wait()

    return run(z, table)


def kernel(z, cheb):
    chebT = lax.optimization_barrier(cheb.T)
    flat = _transpose_pad(chebT, cheb[_X - _AT2:])
    return _series_eval(z, flat.reshape(_X, _YCP))
